# baseline (speedup 1.0000x reference)
import functools

import jax
import jax.numpy as jnp
from jax import lax
from jax.experimental import pallas as pl
from jax.experimental.pallas import tpu as pltpu

F32 = jnp.float32
K_NN = 16
TEMP = 0.7
LN_EPS = 1e-5
RB = 256
RB_T = 512


def _d2_block(pr, pc):
    g = lax.dot_general(pr.astype(jnp.bfloat16), pc.astype(jnp.bfloat16),
                        (((1,), (0,)), ((), ())),
                        preferred_element_type=F32)
    xr, yr, zr = pr[:, 0:1], pr[:, 1:2], pr[:, 2:3]
    xc, yc, zc = pc[0:1, :], pc[1:2, :], pc[2:3, :]
    n2r = xr * xr + yr * yr + zr * zr
    n2c = xc * xc + yc * yc + zc * zc
    d2 = (n2r + n2c) - 2.0 * g
    return jnp.maximum(d2, 0.0)


KEY_MASK = 0x7FFFFFFF
KEY_MAX = 2**31 - 1


def _thresh_body(posp_ref, post_ref, t2_ref):
    i = pl.program_id(0)
    n = post_ref.shape[1]
    d2 = _d2_block(posp_ref[...], post_ref[...])
    colv = lax.broadcasted_iota(jnp.int32, (RB_T, n), 1)
    rowg = i * RB_T + lax.broadcasted_iota(jnp.int32, (RB_T, 1), 0)
    key = lax.bitcast_convert_type(d2, jnp.int32) & jnp.int32(KEY_MASK)
    key = jnp.where(colv == rowg, jnp.int32(KEY_MAX), key)
    lo = jnp.min(key, axis=1, keepdims=True)
    c = jnp.sum((key == lo).astype(jnp.int32), axis=1, keepdims=True)

    def body(_, carry):
        lo, c = carry
        nxt = jnp.min(jnp.where(key > lo, key, jnp.int32(KEY_MAX)),
                      axis=1, keepdims=True)
        cn = jnp.sum((key == nxt).astype(jnp.int32), axis=1, keepdims=True)
        adv = c < K_NN
        return jnp.where(adv, nxt, lo), jnp.where(adv, c + cn, c)

    lo, _ = lax.fori_loop(0, K_NN - 1, body, (lo, c))
    t2_ref[...] = jnp.broadcast_to(lo, (RB_T, 128))


def _attn_block(blk, q, k, v, res, pospb, post, t2rb, t2c, tau, lns, lnb):
    n, d = k.shape
    d2 = _d2_block(pospb, post)
    colv = lax.broadcasted_iota(jnp.int32, (RB, n), 1)
    rowg = blk * RB + lax.broadcasted_iota(jnp.int32, (RB, 1), 0)
    bits = lax.bitcast_convert_type(d2, jnp.int32) & jnp.int32(KEY_MASK)
    m = ((bits <= t2rb[:, 0:1]) & (bits <= t2c[0:1, :])
         & (colv != rowg))
    s = lax.dot_general(q, k, (((1,), (1,)), ((), ())),
                        preferred_element_type=F32)
    e = s * (1.0 / (d ** 0.5))
    g = jax.nn.sigmoid((e - tau) * (1.0 / TEMP))
    g = jnp.where(m, g, 0.0)
    den = jnp.maximum(jnp.sum(g, axis=1, keepdims=True), 1e-6)
    agg = jnp.dot(g, v, preferred_element_type=F32)
    out = res + agg / den
    mu = jnp.mean(out, axis=1, keepdims=True)
    var = jnp.mean((out - mu) ** 2, axis=1, keepdims=True)
    y = (out - mu) * lax.rsqrt(var + LN_EPS) * lns + lnb
    return jnp.maximum(y, 0.0)


def _fused_body(nb, x_ref, posp_ref, post_ref, t2r_ref, t2c_ref,
                wq1, bq1, wk1, bk1, wv1, bv1, wres1, ln1s, ln1b,
                wq2, bq2, wk2, bk2, wv2, bv2, ln2s, ln2b, tau_ref,
                out_ref, k1s, v1s, x1s, k2s, v2s):
    i = pl.program_id(0)
    phase = i // nb
    blk = i % nb
    sl = pl.ds(blk * RB, RB)

    @pl.when(phase == 0)
    def _():
        xb = x_ref[...]
        k1s[sl, :] = jnp.dot(xb, wk1[...], preferred_element_type=F32) + bk1[...]
        v1s[sl, :] = jnp.dot(xb, wv1[...], preferred_element_type=F32) + bv1[...]

    @pl.when(phase == 1)
    def _():
        xb = x_ref[...]
        q = jnp.dot(xb, wq1[...], preferred_element_type=F32) + bq1[...]
        res = jnp.dot(xb, wres1[...], preferred_element_type=F32)
        x1b = _attn_block(blk, q, k1s[...], v1s[...], res, posp_ref[...],
                          post_ref[...], t2r_ref[...], t2c_ref[...],
                          tau_ref[0, 0], ln1s[...], ln1b[...])
        x1s[sl, :] = x1b
        k2s[sl, :] = jnp.dot(x1b, wk2[...], preferred_element_type=F32) + bk2[...]
        v2s[sl, :] = jnp.dot(x1b, wv2[...], preferred_element_type=F32) + bv2[...]

    @pl.when(phase == 2)
    def _():
        x1b = x1s[sl, :]
        q = jnp.dot(x1b, wq2[...], preferred_element_type=F32) + bq2[...]
        out_ref[...] = _attn_block(blk, q, k2s[...], v2s[...], x1b,
                                   posp_ref[...], post_ref[...],
                                   t2r_ref[...], t2c_ref[...],
                                   tau_ref[0, 0], ln2s[...], ln2b[...])


def _row_spec(c):
    return pl.BlockSpec((RB, c), lambda i: (i, 0))


def _rowmod_spec(nb, c):
    return pl.BlockSpec((RB, c), lambda i: (lax.rem(i, nb), 0))


def _full_spec(r, c):
    return pl.BlockSpec((r, c), lambda i: (0, 0))


def kernel(x, pos, Wq1, bq1, Wk1, bk1, Wv1, bv1, Wres1, ln1_s, ln1_b,
           Wq2, bq2, Wk2, bk2, Wv2, bv2, ln2_s, ln2_b, tau):
    n, din = x.shape
    h = Wq1.shape[1]
    nb = n // RB
    posp = jnp.zeros((n, 128), F32).at[:, :3].set(pos)
    post = jnp.zeros((128, n), F32).at[:3, :].set(pos.T)

    t2r = pl.pallas_call(
        _thresh_body,
        grid=(n // RB_T,),
        in_specs=[pl.BlockSpec((RB_T, 128), lambda i: (i, 0)),
                  _full_spec(128, n)],
        out_specs=pl.BlockSpec((RB_T, 128), lambda i: (i, 0)),
        out_shape=jax.ShapeDtypeStruct((n, 128), jnp.int32),
    )(posp, post)
    t2c = jnp.broadcast_to(t2r[:, 0][None, :], (8, n))

    tau2d = tau.reshape(1, 1)
    r1 = lambda a: a.reshape(1, -1)

    x2 = pl.pallas_call(
        functools.partial(_fused_body, nb),
        grid=(3 * nb,),
        in_specs=[
            _rowmod_spec(nb, din), _rowmod_spec(nb, 128), _full_spec(128, n),
            _rowmod_spec(nb, 128), _full_spec(8, n),
            _full_spec(din, h), _full_spec(1, h),
            _full_spec(din, h), _full_spec(1, h),
            _full_spec(din, h), _full_spec(1, h),
            _full_spec(din, h),
            _full_spec(1, h), _full_spec(1, h),
            _full_spec(h, h), _full_spec(1, h),
            _full_spec(h, h), _full_spec(1, h),
            _full_spec(h, h), _full_spec(1, h),
            _full_spec(1, h), _full_spec(1, h),
            pl.BlockSpec(memory_space=pltpu.SMEM),
        ],
        out_specs=pl.BlockSpec(
            (RB, h), lambda i: (jnp.where(i < 2 * nb, 0, i - 2 * nb), 0)),
        out_shape=jax.ShapeDtypeStruct((n, h), F32),
        scratch_shapes=[pltpu.VMEM((n, h), F32)] * 5,
    )(x, posp, post, t2r, t2c,
      Wq1, r1(bq1), Wk1, r1(bk1), Wv1, r1(bv1), Wres1, r1(ln1_s), r1(ln1_b),
      Wq2, r1(bq2), Wk2, r1(bk2), Wv2, r1(bv2), r1(ln2_s), r1(ln2_b), tau2d)
    return x2


# speedup vs baseline: 1.0099x; 1.0099x over previous
import functools

import jax
import jax.numpy as jnp
from jax import lax
from jax.experimental import pallas as pl
from jax.experimental.pallas import tpu as pltpu

F32 = jnp.float32
K_NN = 16
TEMP = 0.7
LN_EPS = 1e-5
RB = 256
RB_T = 1024


def _d2_block(pr, pc):
    g = lax.dot_general(pr.astype(jnp.bfloat16), pc.astype(jnp.bfloat16),
                        (((1,), (0,)), ((), ())),
                        preferred_element_type=F32)
    xr, yr, zr = pr[:, 0:1], pr[:, 1:2], pr[:, 2:3]
    xc, yc, zc = pc[0:1, :], pc[1:2, :], pc[2:3, :]
    n2r = xr * xr + yr * yr + zr * zr
    n2c = xc * xc + yc * yc + zc * zc
    d2 = (n2r + n2c) - 2.0 * g
    return jnp.maximum(d2, 0.0)


KEY_MASK = 0x7FFFFFFF
KEY_MAX = 2**31 - 1


def _thresh_body(posp_ref, post_ref, t2_ref):
    i = pl.program_id(0)
    n = post_ref.shape[1]
    d2 = _d2_block(posp_ref[...], post_ref[...])
    colv = lax.broadcasted_iota(jnp.int32, (RB_T, n), 1)
    rowg = i * RB_T + lax.broadcasted_iota(jnp.int32, (RB_T, 1), 0)
    key = lax.bitcast_convert_type(d2, jnp.int32) & jnp.int32(KEY_MASK)
    key = jnp.where(colv == rowg, jnp.int32(KEY_MAX), key)
    lo = jnp.min(key, axis=1, keepdims=True)
    c = jnp.sum((key == lo).astype(jnp.int32), axis=1, keepdims=True)

    def body(_, carry):
        lo, c = carry
        nxt = jnp.min(jnp.where(key > lo, key, jnp.int32(KEY_MAX)),
                      axis=1, keepdims=True)
        cn = jnp.sum((key == nxt).astype(jnp.int32), axis=1, keepdims=True)
        adv = c < K_NN
        return jnp.where(adv, nxt, lo), jnp.where(adv, c + cn, c)

    lo, _ = lax.fori_loop(0, K_NN - 1, body, (lo, c))
    t2_ref[...] = jnp.broadcast_to(lo, (RB_T, 128))


def _attn_block(blk, q, k, v, res, pospb, post, t2rb, t2c, tau, lns, lnb):
    n, d = k.shape
    d2 = _d2_block(pospb, post)
    colv = lax.broadcasted_iota(jnp.int32, (RB, n), 1)
    rowg = blk * RB + lax.broadcasted_iota(jnp.int32, (RB, 1), 0)
    bits = lax.bitcast_convert_type(d2, jnp.int32) & jnp.int32(KEY_MASK)
    m = ((bits <= t2rb[:, 0:1]) & (bits <= t2c[0:1, :])
         & (colv != rowg))
    s = lax.dot_general(q, k, (((1,), (1,)), ((), ())),
                        preferred_element_type=F32)
    e = s * (1.0 / (d ** 0.5))
    g = jax.nn.sigmoid((e - tau) * (1.0 / TEMP))
    g = jnp.where(m, g, 0.0)
    den = jnp.maximum(jnp.sum(g, axis=1, keepdims=True), 1e-6)
    agg = jnp.dot(g, v, preferred_element_type=F32)
    out = res + agg / den
    mu = jnp.mean(out, axis=1, keepdims=True)
    var = jnp.mean((out - mu) ** 2, axis=1, keepdims=True)
    y = (out - mu) * lax.rsqrt(var + LN_EPS) * lns + lnb
    return jnp.maximum(y, 0.0)


def _fused_body(nb, x_ref, posp_ref, post_ref, t2r_ref, t2c_ref,
                wq1, bq1, wk1, bk1, wv1, bv1, wres1, ln1s, ln1b,
                wq2, bq2, wk2, bk2, wv2, bv2, ln2s, ln2b, tau_ref,
                out_ref, k1s, v1s, x1s, k2s, v2s):
    i = pl.program_id(0)
    phase = i // nb
    blk = i % nb
    sl = pl.ds(blk * RB, RB)

    @pl.when(phase == 0)
    def _():
        xb = x_ref[...]
        k1s[sl, :] = jnp.dot(xb, wk1[...], preferred_element_type=F32) + bk1[...]
        v1s[sl, :] = jnp.dot(xb, wv1[...], preferred_element_type=F32) + bv1[...]

    @pl.when(phase == 1)
    def _():
        xb = x_ref[...]
        q = jnp.dot(xb, wq1[...], preferred_element_type=F32) + bq1[...]
        res = jnp.dot(xb, wres1[...], preferred_element_type=F32)
        x1b = _attn_block(blk, q, k1s[...], v1s[...], res, posp_ref[...],
                          post_ref[...], t2r_ref[...], t2c_ref[...],
                          tau_ref[0, 0], ln1s[...], ln1b[...])
        x1s[sl, :] = x1b
        k2s[sl, :] = jnp.dot(x1b, wk2[...], preferred_element_type=F32) + bk2[...]
        v2s[sl, :] = jnp.dot(x1b, wv2[...], preferred_element_type=F32) + bv2[...]

    @pl.when(phase == 2)
    def _():
        x1b = x1s[sl, :]
        q = jnp.dot(x1b, wq2[...], preferred_element_type=F32) + bq2[...]
        out_ref[...] = _attn_block(blk, q, k2s[...], v2s[...], x1b,
                                   posp_ref[...], post_ref[...],
                                   t2r_ref[...], t2c_ref[...],
                                   tau_ref[0, 0], ln2s[...], ln2b[...])


def _row_spec(c):
    return pl.BlockSpec((RB, c), lambda i: (i, 0))


def _rowmod_spec(nb, c):
    return pl.BlockSpec((RB, c), lambda i: (lax.rem(i, nb), 0))


def _full_spec(r, c):
    return pl.BlockSpec((r, c), lambda i: (0, 0))


def kernel(x, pos, Wq1, bq1, Wk1, bk1, Wv1, bv1, Wres1, ln1_s, ln1_b,
           Wq2, bq2, Wk2, bk2, Wv2, bv2, ln2_s, ln2_b, tau):
    n, din = x.shape
    h = Wq1.shape[1]
    nb = n // RB
    posp = jnp.zeros((n, 128), F32).at[:, :3].set(pos)
    post = jnp.zeros((128, n), F32).at[:3, :].set(pos.T)

    t2r = pl.pallas_call(
        _thresh_body,
        grid=(n // RB_T,),
        in_specs=[pl.BlockSpec((RB_T, 128), lambda i: (i, 0)),
                  _full_spec(128, n)],
        out_specs=pl.BlockSpec((RB_T, 128), lambda i: (i, 0)),
        out_shape=jax.ShapeDtypeStruct((n, 128), jnp.int32),
    )(posp, post)
    t2c = jnp.broadcast_to(t2r[:, 0][None, :], (8, n))

    tau2d = tau.reshape(1, 1)
    r1 = lambda a: a.reshape(1, -1)

    x2 = pl.pallas_call(
        functools.partial(_fused_body, nb),
        grid=(3 * nb,),
        in_specs=[
            _rowmod_spec(nb, din), _rowmod_spec(nb, 128), _full_spec(128, n),
            _rowmod_spec(nb, 128), _full_spec(8, n),
            _full_spec(din, h), _full_spec(1, h),
            _full_spec(din, h), _full_spec(1, h),
            _full_spec(din, h), _full_spec(1, h),
            _full_spec(din, h),
            _full_spec(1, h), _full_spec(1, h),
            _full_spec(h, h), _full_spec(1, h),
            _full_spec(h, h), _full_spec(1, h),
            _full_spec(h, h), _full_spec(1, h),
            _full_spec(1, h), _full_spec(1, h),
            pl.BlockSpec(memory_space=pltpu.SMEM),
        ],
        out_specs=pl.BlockSpec(
            (RB, h), lambda i: (jnp.where(i < 2 * nb, 0, i - 2 * nb), 0)),
        out_shape=jax.ShapeDtypeStruct((n, h), F32),
        scratch_shapes=[pltpu.VMEM((n, h), F32)] * 5,
    )(x, posp, post, t2r, t2c,
      Wq1, r1(bq1), Wk1, r1(bk1), Wv1, r1(bv1), Wres1, r1(ln1_s), r1(ln1_b),
      Wq2, r1(bq2), Wk2, r1(bk2), Wv2, r1(bv2), r1(ln2_s), r1(ln2_b), tau2d)
    return x2
